# baseline (speedup 1.0000x reference)
import jax
import jax.numpy as jnp
from jax import lax
from jax.experimental import pallas as pl
from jax.experimental.pallas import tpu as pltpu
from jax.experimental.pallas import tpu_sc as plsc

D_MODEL = 256
HALF = 128
N_PAT = 64
LANES = 16

B, H, W = 128, 30, 30
N = B * H * W
NW = 32
PER_W = N // NW
CHUNK = 240
NCHUNK = PER_W // CHUNK
NBUF = 3
GRP = CHUNK // LANES


def _gather_body(idx_hbm, ppe_hbm, out_hbm, tbl_v, pt0, pt1, pt2, pti_v,
                 so0, so1, so2):
    pt_bufs = (pt0, pt1, pt2)
    sem_out = (so0, so1, so2)
    wid = lax.axis_index("s") * 2 + lax.axis_index("c")
    base = wid * PER_W
    iota = lax.iota(jnp.int32, LANES)

    @pl.when(lax.axis_index("s") == 0)
    def _stage_table():
        pltpu.sync_copy(ppe_hbm, tbl_v)

    pltpu.sync_copy(idx_hbm.at[pl.ds(base, PER_W)], pti_v)
    plsc.subcore_barrier()

    def prep(g, t):
        sl = pl.ds(g * LANES, LANES)
        pti_v[sl] = lax.bitwise_and(pti_v[sl], N_PAT - 1)
        return t

    lax.fori_loop(0, PER_W // LANES, prep, 0)

    def compute(c, b):
        cp = pltpu.async_copy(
            tbl_v.at[pti_v.at[pl.ds(c * CHUNK, CHUNK)]], pt_bufs[b],
            sem_out[b])
        cp.wait()

    def issue_out(c, b):
        pltpu.async_copy(pt_bufs[b], out_hbm.at[pl.ds(base + c * CHUNK, CHUNK)],
                         sem_out[b])

    def wait_out(b):
        pltpu.make_async_copy(pt_bufs[b], out_hbm.at[pl.ds(0, CHUNK)],
                              sem_out[b]).wait()

    for k in range(NBUF):
        compute(k, k)
        issue_out(k, k)

    def outer(co, t):
        for k in range(NBUF):
            c = NBUF * co + k
            wait_out(k)
            compute(c, k)
            issue_out(c, k)
        return t

    lax.fori_loop(1, NCHUNK // NBUF, outer, 0)
    wait_out(0)
    wait_out(1)
    wait_out(2)


def _sc_gather(idxf, pattern_pe):
    mesh = plsc.VectorSubcoreMesh(core_axis_name="c", subcore_axis_name="s")
    return pl.kernel(
        _gather_body,
        out_type=jax.ShapeDtypeStruct((N, HALF), jnp.float32),
        mesh=mesh,
        scratch_types=[
            pltpu.VMEM_SHARED((N_PAT, HALF), jnp.float32),
            pltpu.VMEM((CHUNK, HALF), jnp.float32),
            pltpu.VMEM((CHUNK, HALF), jnp.float32),
            pltpu.VMEM((CHUNK, HALF), jnp.float32),
            pltpu.VMEM((PER_W,), jnp.int32),
            pltpu.SemaphoreType.DMA,
            pltpu.SemaphoreType.DMA,
            pltpu.SemaphoreType.DMA,
        ],
    )(idxf, pattern_pe)


def _add_body(x_ref, sp_ref, pc_ref, out_ref):
    sp = sp_ref[...]
    pc = pc_ref[...]
    out_ref[..., :HALF] = x_ref[..., :HALF] + sp[:, :, None, :]
    out_ref[..., HALF:] = x_ref[..., HALF:] + pc.reshape(1, W, B, HALF)


def _tc_add(xT, spatial_pe, penc):
    return pl.pallas_call(
        _add_body,
        grid=(H,),
        in_specs=[
            pl.BlockSpec((1, W, B, D_MODEL), lambda i: (i, 0, 0, 0)),
            pl.BlockSpec((1, W, HALF), lambda i: (i, 0, 0)),
            pl.BlockSpec((W * B, HALF), lambda i: (i, 0)),
        ],
        out_specs=pl.BlockSpec((1, W, B, D_MODEL), lambda i: (i, 0, 0, 0)),
        out_shape=jax.ShapeDtypeStruct((H, W, B, D_MODEL), jnp.float32),
    )(xT, spatial_pe, penc)


@jax.jit
def kernel(x, pattern_indices, spatial_pe, pattern_pe):
    xT = jnp.transpose(x, (1, 2, 0, 3))
    idxT = jnp.transpose(pattern_indices, (1, 2, 0)).reshape(N).astype(jnp.int32)
    penc = _sc_gather(idxT, pattern_pe)
    outT = _tc_add(xT, spatial_pe, penc)
    return jnp.transpose(outT, (2, 0, 1, 3))


# speedup vs baseline: 1.0001x; 1.0001x over previous
import jax
import jax.numpy as jnp
from jax import lax
from jax.experimental import pallas as pl
from jax.experimental.pallas import tpu as pltpu
from jax.experimental.pallas import tpu_sc as plsc

D_MODEL = 256
HALF = 128
N_PAT = 64
LANES = 16

B, H, W = 128, 30, 30
N = B * H * W
NW = 32
PER_W = N // NW
CHUNK = 240
NCHUNK = PER_W // CHUNK
NBUF = 3


def _gather_body(idx_hbm, ppe_hbm, out_hbm, tbl_v, pt0, pt1, pt2, pti_v,
                 so0, so1, so2):
    pt_bufs = (pt0, pt1, pt2)
    sem_out = (so0, so1, so2)
    wid = lax.axis_index("s") * 2 + lax.axis_index("c")
    base = wid * PER_W

    @pl.when(lax.axis_index("s") == 0)
    def _stage_table():
        pltpu.sync_copy(ppe_hbm, tbl_v)

    pltpu.sync_copy(idx_hbm.at[pl.ds(base, PER_W)], pti_v)
    plsc.subcore_barrier()

    def prep(g, t):
        sl = pl.ds(g * LANES, LANES)
        pti_v[sl] = lax.bitwise_and(pti_v[sl], N_PAT - 1)
        return t

    lax.fori_loop(0, PER_W // LANES, prep, 0)

    def compute(c, b):
        cp = pltpu.async_copy(
            tbl_v.at[pti_v.at[pl.ds(c * CHUNK, CHUNK)]], pt_bufs[b],
            sem_out[b])
        cp.wait()

    def issue_out(c, b):
        pltpu.async_copy(pt_bufs[b], out_hbm.at[pl.ds(base + c * CHUNK, CHUNK)],
                         sem_out[b])

    def wait_out(b):
        pltpu.make_async_copy(pt_bufs[b], out_hbm.at[pl.ds(0, CHUNK)],
                              sem_out[b]).wait()

    for k in range(NBUF):
        compute(k, k)
        issue_out(k, k)

    def outer(co, t):
        for k in range(NBUF):
            c = NBUF * co + k
            wait_out(k)
            compute(c, k)
            issue_out(c, k)
        return t

    lax.fori_loop(1, NCHUNK // NBUF, outer, 0)
    wait_out(0)
    wait_out(1)
    wait_out(2)


def _sc_gather(idxf, pattern_pe):
    mesh = plsc.VectorSubcoreMesh(core_axis_name="c", subcore_axis_name="s")
    return pl.kernel(
        _gather_body,
        out_type=jax.ShapeDtypeStruct((N, HALF), jnp.float32),
        mesh=mesh,
        scratch_types=[
            pltpu.VMEM_SHARED((N_PAT, HALF), jnp.float32),
            pltpu.VMEM((CHUNK, HALF), jnp.float32),
            pltpu.VMEM((CHUNK, HALF), jnp.float32),
            pltpu.VMEM((CHUNK, HALF), jnp.float32),
            pltpu.VMEM((PER_W,), jnp.int32),
            pltpu.SemaphoreType.DMA,
            pltpu.SemaphoreType.DMA,
            pltpu.SemaphoreType.DMA,
        ],
    )(idxf, pattern_pe)


HB = 2


def _add_body(x_ref, sp_ref, pc_ref, out_ref):
    sp = sp_ref[...]
    pc = pc_ref[...]
    out_ref[..., :HALF] = x_ref[..., :HALF] + sp[:, :, None, :]
    out_ref[..., HALF:] = x_ref[..., HALF:] + pc.reshape(HB, W, B, HALF)


def _tc_add(xT, spatial_pe, penc):
    return pl.pallas_call(
        _add_body,
        grid=(H // HB,),
        in_specs=[
            pl.BlockSpec((HB, W, B, D_MODEL), lambda i: (i, 0, 0, 0)),
            pl.BlockSpec((HB, W, HALF), lambda i: (i, 0, 0)),
            pl.BlockSpec((HB * W * B, HALF), lambda i: (i, 0)),
        ],
        out_specs=pl.BlockSpec((HB, W, B, D_MODEL), lambda i: (i, 0, 0, 0)),
        out_shape=jax.ShapeDtypeStruct((H, W, B, D_MODEL), jnp.float32),
    )(xT, spatial_pe, penc)


@jax.jit
def kernel(x, pattern_indices, spatial_pe, pattern_pe):
    xT = jnp.transpose(x, (1, 2, 0, 3))
    idxT = jnp.transpose(pattern_indices, (1, 2, 0)).reshape(N).astype(jnp.int32)
    penc = _sc_gather(idxT, pattern_pe)
    outT = _tc_add(xT, spatial_pe, penc)
    return jnp.transpose(outT, (2, 0, 1, 3))


# speedup vs baseline: 1.0055x; 1.0054x over previous
import jax
import jax.numpy as jnp
from jax import lax
from jax.experimental import pallas as pl
from jax.experimental.pallas import tpu as pltpu
from jax.experimental.pallas import tpu_sc as plsc

D_MODEL = 256
HALF = 128
N_PAT = 64
LANES = 16

B, H, W = 128, 30, 30
N = B * H * W
NW = 32
PER_W = N // NW
CHUNK = 240
NCHUNK = PER_W // CHUNK
NBUF = 3


def _gather_body(idx_hbm, ppe_hbm, out_hbm, tbl_v, pt0, pt1, pt2, pti_v,
                 si0, si1, si2, so0, so1, so2):
    pt_bufs = (pt0, pt1, pt2)
    sem_in = (si0, si1, si2)
    sem_out = (so0, so1, so2)
    wid = lax.axis_index("s") * 2 + lax.axis_index("c")
    base = wid * PER_W

    @pl.when(lax.axis_index("s") == 0)
    def _stage_table():
        pltpu.sync_copy(ppe_hbm, tbl_v)

    pltpu.sync_copy(idx_hbm.at[pl.ds(base, PER_W)], pti_v)
    plsc.subcore_barrier()

    def prep(g, t):
        sl = pl.ds(g * LANES, LANES)
        pti_v[sl] = lax.bitwise_and(pti_v[sl], N_PAT - 1)
        return t

    lax.fori_loop(0, PER_W // LANES, prep, 0)

    def issue_gather(c, b):
        pltpu.async_copy(tbl_v.at[pti_v.at[pl.ds(c * CHUNK, CHUNK)]],
                         pt_bufs[b], sem_in[b])

    def wait_gather(b):
        pltpu.make_async_copy(out_hbm.at[pl.ds(0, CHUNK)], pt_bufs[b],
                              sem_in[b]).wait()

    def issue_out(c, b):
        pltpu.async_copy(pt_bufs[b], out_hbm.at[pl.ds(base + c * CHUNK, CHUNK)],
                         sem_out[b])

    def wait_out(b):
        pltpu.make_async_copy(pt_bufs[b], out_hbm.at[pl.ds(0, CHUNK)],
                              sem_out[b]).wait()

    issue_gather(0, 0)
    issue_gather(1, 1)
    issue_gather(2, 2)
    wait_gather(0); issue_out(0, 0)
    wait_out(0); issue_gather(3, 0)
    wait_gather(1); issue_out(1, 1)
    wait_out(1); issue_gather(4, 1)
    wait_gather(2); issue_out(2, 2)

    def outer(co, t):
        for k in range(NBUF):
            c = NBUF * co + k
            pb = (k + 2) % NBUF

            @pl.when(c + 2 < NCHUNK)
            def _prefetch():
                wait_out(pb)
                issue_gather(c + 2, pb)

            wait_gather(k)
            issue_out(c, k)
        return t

    lax.fori_loop(1, NCHUNK // NBUF, outer, 0)
    wait_out(0)
    wait_out(1)
    wait_out(2)


def _sc_gather(idxf, pattern_pe):
    mesh = plsc.VectorSubcoreMesh(core_axis_name="c", subcore_axis_name="s")
    return pl.kernel(
        _gather_body,
        out_type=jax.ShapeDtypeStruct((N, HALF), jnp.float32),
        mesh=mesh,
        scratch_types=[
            pltpu.VMEM_SHARED((N_PAT, HALF), jnp.float32),
            pltpu.VMEM((CHUNK, HALF), jnp.float32),
            pltpu.VMEM((CHUNK, HALF), jnp.float32),
            pltpu.VMEM((CHUNK, HALF), jnp.float32),
            pltpu.VMEM((PER_W,), jnp.int32),
            pltpu.SemaphoreType.DMA,
            pltpu.SemaphoreType.DMA,
            pltpu.SemaphoreType.DMA,
            pltpu.SemaphoreType.DMA,
            pltpu.SemaphoreType.DMA,
            pltpu.SemaphoreType.DMA,
        ],
    )(idxf, pattern_pe)


HB = 2


def _add_body(x_ref, sp_ref, pc_ref, out_ref):
    sp = sp_ref[...]
    pc = pc_ref[...]
    out_ref[..., :HALF] = x_ref[..., :HALF] + sp[:, :, None, :]
    out_ref[..., HALF:] = x_ref[..., HALF:] + pc.reshape(HB, W, B, HALF)


def _tc_add(xT, spatial_pe, penc):
    return pl.pallas_call(
        _add_body,
        grid=(H // HB,),
        in_specs=[
            pl.BlockSpec((HB, W, B, D_MODEL), lambda i: (i, 0, 0, 0)),
            pl.BlockSpec((HB, W, HALF), lambda i: (i, 0, 0)),
            pl.BlockSpec((HB * W * B, HALF), lambda i: (i, 0)),
        ],
        out_specs=pl.BlockSpec((HB, W, B, D_MODEL), lambda i: (i, 0, 0, 0)),
        out_shape=jax.ShapeDtypeStruct((H, W, B, D_MODEL), jnp.float32),
    )(xT, spatial_pe, penc)


@jax.jit
def kernel(x, pattern_indices, spatial_pe, pattern_pe):
    xT = jnp.transpose(x, (1, 2, 0, 3))
    idxT = jnp.transpose(pattern_indices, (1, 2, 0)).reshape(N).astype(jnp.int32)
    penc = _sc_gather(idxT, pattern_pe)
    outT = _tc_add(xT, spatial_pe, penc)
    return jnp.transpose(outT, (2, 0, 1, 3))


# speedup vs baseline: 1.0077x; 1.0023x over previous
import jax
import jax.numpy as jnp
from jax import lax
from jax.experimental import pallas as pl
from jax.experimental.pallas import tpu as pltpu
from jax.experimental.pallas import tpu_sc as plsc

D_MODEL = 256
HALF = 128
N_PAT = 64
LANES = 16

B, H, W = 128, 30, 30
N = B * H * W
NW = 32
PER_W = N // NW
CHUNK = 240
NCHUNK = PER_W // CHUNK
NBUF = 3


def _gather_body(idx_hbm, ppe_hbm, out_hbm, tbl_v, pt0, pt1, pt2, pti_v,
                 si0, si1, si2, so0, so1, so2):
    pt_bufs = (pt0, pt1, pt2)
    sem_in = (si0, si1, si2)
    sem_out = (so0, so1, so2)
    wid = lax.axis_index("s") * 2 + lax.axis_index("c")
    base = wid * PER_W

    @pl.when(lax.axis_index("s") == 0)
    def _stage_table():
        pltpu.sync_copy(ppe_hbm, tbl_v)

    pltpu.sync_copy(idx_hbm.at[pl.ds(base, PER_W)], pti_v)
    plsc.subcore_barrier()

    def prep(g, t):
        sl = pl.ds(g * LANES, LANES)
        pti_v[sl] = lax.bitwise_and(pti_v[sl], N_PAT - 1)
        return t

    lax.fori_loop(0, PER_W // LANES, prep, 0)

    def issue_gather(c, b):
        pltpu.async_copy(tbl_v.at[pti_v.at[pl.ds(c * CHUNK, CHUNK)]],
                         pt_bufs[b], sem_in[b])

    def wait_gather(b):
        pltpu.make_async_copy(out_hbm.at[pl.ds(0, CHUNK)], pt_bufs[b],
                              sem_in[b]).wait()

    def issue_out(c, b):
        pltpu.async_copy(pt_bufs[b], out_hbm.at[pl.ds(base + c * CHUNK, CHUNK)],
                         sem_out[b])

    def wait_out(b):
        pltpu.make_async_copy(pt_bufs[b], out_hbm.at[pl.ds(0, CHUNK)],
                              sem_out[b]).wait()

    issue_gather(0, 0)
    issue_gather(1, 1)
    issue_gather(2, 2)
    wait_gather(0); issue_out(0, 0)
    wait_out(0); issue_gather(3, 0)
    wait_gather(1); issue_out(1, 1)
    wait_out(1); issue_gather(4, 1)
    wait_gather(2); issue_out(2, 2)

    def outer(co, t):
        for k in range(NBUF):
            c = NBUF * co + k
            pb = (k + 2) % NBUF

            @pl.when(c + 2 < NCHUNK)
            def _prefetch():
                wait_out(pb)
                issue_gather(c + 2, pb)

            wait_gather(k)
            issue_out(c, k)
        return t

    lax.fori_loop(1, NCHUNK // NBUF, outer, 0)
    wait_out(0)
    wait_out(1)
    wait_out(2)


def _sc_gather(idxf, pattern_pe):
    mesh = plsc.VectorSubcoreMesh(core_axis_name="c", subcore_axis_name="s")
    return pl.kernel(
        _gather_body,
        out_type=jax.ShapeDtypeStruct((N, HALF), jnp.float32),
        mesh=mesh,
        scratch_types=[
            pltpu.VMEM_SHARED((N_PAT, HALF), jnp.float32),
            pltpu.VMEM((CHUNK, HALF), jnp.float32),
            pltpu.VMEM((CHUNK, HALF), jnp.float32),
            pltpu.VMEM((CHUNK, HALF), jnp.float32),
            pltpu.VMEM((PER_W,), jnp.int32),
            pltpu.SemaphoreType.DMA,
            pltpu.SemaphoreType.DMA,
            pltpu.SemaphoreType.DMA,
            pltpu.SemaphoreType.DMA,
            pltpu.SemaphoreType.DMA,
            pltpu.SemaphoreType.DMA,
        ],
    )(idxf, pattern_pe)


HB = 1


def _add_body(x_ref, sp_ref, pc_ref, out_ref):
    sp = sp_ref[...]
    pc = pc_ref[...]
    out_ref[..., :HALF] = x_ref[..., :HALF] + sp[:, :, None, :]
    out_ref[..., HALF:] = x_ref[..., HALF:] + pc.reshape(HB, W, B, HALF)


def _tc_add(xT, spatial_pe, penc):
    return pl.pallas_call(
        _add_body,
        grid=(H // HB,),
        in_specs=[
            pl.BlockSpec((HB, W, B, D_MODEL), lambda i: (i, 0, 0, 0)),
            pl.BlockSpec((HB, W, HALF), lambda i: (i, 0, 0)),
            pl.BlockSpec((HB * W * B, HALF), lambda i: (i, 0)),
        ],
        out_specs=pl.BlockSpec((HB, W, B, D_MODEL), lambda i: (i, 0, 0, 0)),
        out_shape=jax.ShapeDtypeStruct((H, W, B, D_MODEL), jnp.float32),
    )(xT, spatial_pe, penc)


@jax.jit
def kernel(x, pattern_indices, spatial_pe, pattern_pe):
    xT = jnp.transpose(x, (1, 2, 0, 3))
    idxT = jnp.transpose(pattern_indices, (1, 2, 0)).reshape(N).astype(jnp.int32)
    penc = _sc_gather(idxT, pattern_pe)
    outT = _tc_add(xT, spatial_pe, penc)
    return jnp.transpose(outT, (2, 0, 1, 3))
